# Initial kernel scaffold; baseline (speedup 1.0000x reference)
import jax
import jax.numpy as jnp
from jax.experimental import pallas as pl


def kernel(x, Wg, bg, W1, b1, W2, b2, gamma, beta):
    raise NotImplementedError("write your pallas kernel here")


# speedup vs baseline: 3.3943x; 3.3943x over previous
import jax
import jax.numpy as jnp
from jax.experimental import pallas as pl

NUM_EXPERTS = 8
TOP_K = 2
ALPHA = 0.5
D_MODEL = 768
D_HIDDEN = 512
SEQ = 2048
TB = 512


def _moe_block(x_ref, Wg_ref, bg_ref, W1_ref, b1_ref, W2_ref, b2_ref,
               gamma_ref, beta_ref, o_ref):
    xb = x_ref[...]
    logits = jnp.dot(xb, Wg_ref[...],
                     preferred_element_type=jnp.float32) + bg_ref[...]
    m = jnp.max(logits, axis=-1, keepdims=True)
    p = jnp.exp(logits - m)
    p = p / jnp.sum(p, axis=-1, keepdims=True)
    eidx = jax.lax.broadcasted_iota(jnp.int32, (TB, NUM_EXPERTS), 1)
    m0 = jnp.max(p, axis=-1, keepdims=True)
    e0 = jnp.min(jnp.where(p == m0, eidx, NUM_EXPERTS), axis=-1,
                 keepdims=True)
    p_m = jnp.where(eidx == e0, -jnp.inf, p)
    m1 = jnp.max(p_m, axis=-1, keepdims=True)
    e1 = jnp.min(jnp.where(p_m == m1, eidx, NUM_EXPERTS), axis=-1,
                 keepdims=True)

    def selected_mlp(inp, e_sel):
        out = jnp.zeros((TB, D_MODEL), jnp.float32)
        for e in range(NUM_EXPERTS):
            h = jnp.maximum(
                jnp.dot(inp, W1_ref[e], preferred_element_type=jnp.float32)
                + b1_ref[e], 0.0)
            oe = jnp.dot(h, W2_ref[e],
                         preferred_element_type=jnp.float32) + b2_ref[e]
            out = jnp.where(e_sel == e, oe, out)
        return out

    out0 = selected_mlp(xb, e0)
    out1 = selected_mlp(xb + ALPHA * out0, e1)
    fused = m0 * out0 + m1 * out1
    y = xb + fused
    mu = jnp.mean(y, axis=-1, keepdims=True)
    yc = y - mu
    var = jnp.mean(yc * yc, axis=-1, keepdims=True)
    o_ref[...] = yc * jax.lax.rsqrt(var + 1e-5) * gamma_ref[...] + beta_ref[...]


def kernel(x, Wg, bg, W1, b1, W2, b2, gamma, beta):
    x2 = x.reshape(SEQ, D_MODEL)
    grid = (SEQ // TB,)
    const = lambda i: (0,) * 1
    out = pl.pallas_call(
        _moe_block,
        grid=grid,
        in_specs=[
            pl.BlockSpec((TB, D_MODEL), lambda i: (i, 0)),
            pl.BlockSpec((D_MODEL, NUM_EXPERTS), lambda i: (0, 0)),
            pl.BlockSpec((NUM_EXPERTS,), lambda i: (0,)),
            pl.BlockSpec((NUM_EXPERTS, D_MODEL, D_HIDDEN), lambda i: (0, 0, 0)),
            pl.BlockSpec((NUM_EXPERTS, D_HIDDEN), lambda i: (0, 0)),
            pl.BlockSpec((NUM_EXPERTS, D_HIDDEN, D_MODEL), lambda i: (0, 0, 0)),
            pl.BlockSpec((NUM_EXPERTS, D_MODEL), lambda i: (0, 0)),
            pl.BlockSpec((D_MODEL,), lambda i: (0,)),
            pl.BlockSpec((D_MODEL,), lambda i: (0,)),
        ],
        out_specs=pl.BlockSpec((TB, D_MODEL), lambda i: (i, 0)),
        out_shape=jax.ShapeDtypeStruct((SEQ, D_MODEL), jnp.float32),
    )(x2, Wg, bg, W1, b1, W2, b2, gamma, beta)
    return out.reshape(1, SEQ, D_MODEL)
